# baseline (speedup 1.0000x reference)
import functools

import jax
import jax.numpy as jnp
from jax import lax
from jax.experimental import pallas as pl
from jax.experimental.pallas import tpu as pltpu
from jax.experimental.pallas import tpu_sc as plsc

_B = 1024
_D = 32
_P = 5
_N = 10
_NROWS = _B * (1 + _P + _N)
_CHUNK = 128

_JB = 1024
_NBLK = (_P + _N) * _B // _JB
_POS_BLKS = _P * _B // _JB


def _gather_rows(idx, table):
    info = plsc.get_sparse_core_info()
    nc, ns = info.num_cores, info.num_subcores
    nw = nc * ns
    rows_per_w = _NROWS // nw
    nchunk = rows_per_w // _CHUNK
    idx2d = idx.reshape(nw * nchunk, _CHUNK)
    mesh = plsc.VectorSubcoreMesh(core_axis_name="c", subcore_axis_name="s")

    @functools.partial(
        pl.kernel,
        mesh=mesh,
        out_type=jax.ShapeDtypeStruct((_NROWS, _D), jnp.float32),
        scratch_types=[
            pltpu.VMEM((nchunk, _CHUNK), jnp.int32),
            pltpu.VMEM((rows_per_w, _D), jnp.float32),
            pltpu.SemaphoreType.DMA,
        ],
        compiler_params=pltpu.CompilerParams(use_tc_tiling_on_sc=False),
    )
    def gather_k(idx_hbm, table_hbm, out_hbm, idx_v, rows_v, sem):
        wid = lax.axis_index("s") * nc + lax.axis_index("c")
        pltpu.sync_copy(idx_hbm.at[pl.ds(wid * nchunk, nchunk)], idx_v)
        copies = [
            pltpu.async_copy(
                table_hbm.at[idx_v.at[c]],
                rows_v.at[pl.ds(c * _CHUNK, _CHUNK)],
                sem,
            )
            for c in range(nchunk)
        ]
        for cp in copies:
            cp.wait()
        pltpu.sync_copy(rows_v, out_hbm.at[pl.ds(wid * rows_per_w, rows_per_w)])

    return gather_k(idx2d, table)


def _cross_body(center_ref, ctx_ref, out_ref):
    j = pl.program_id(0)
    sign = jnp.where(j < _POS_BLKS, jnp.float32(1.0), jnp.float32(-1.0))
    acc = lax.dot_general(
        center_ref[...],
        ctx_ref[...],
        (((1,), (1,)), ((), ())),
        preferred_element_type=jnp.float32,
    )
    out_ref[...] = jax.nn.sigmoid(acc * sign)


def kernel(x_center, x_positive, x_negative, emb_table):
    idx = jnp.concatenate(
        [
            x_center.astype(jnp.int32).reshape(-1),
            x_positive.astype(jnp.int32).T.reshape(-1),
            x_negative.astype(jnp.int32).T.reshape(-1),
        ]
    )
    rows = _gather_rows(idx, emb_table)
    return pl.pallas_call(
        _cross_body,
        grid=(_NBLK,),
        in_specs=[
            pl.BlockSpec((_B, _D), lambda j: (0, 0)),
            pl.BlockSpec((_JB, _D), lambda j: (1 + j, 0)),
        ],
        out_specs=pl.BlockSpec((_B, _JB), lambda j: (0, j)),
        out_shape=jax.ShapeDtypeStruct((_B, (_P + _N) * _B), jnp.float32),
    )(rows, rows)
